# Initial kernel scaffold; baseline (speedup 1.0000x reference)
import jax
import jax.numpy as jnp
from jax.experimental import pallas as pl


def kernel(edge_index, users_w, items_w, W1, b1, W2, b2):
    raise NotImplementedError("write your pallas kernel here")


# speedup vs baseline: 16.2407x; 16.2407x over previous
import functools

import jax
import jax.numpy as jnp
from jax import lax
from jax.experimental import pallas as pl
from jax.experimental.pallas import tpu as pltpu
from jax.experimental.pallas import tpu_sc as plsc

N = 10000
H = 128
E = 320000
NUSERS = 4000

NC = 2
NS = 16
E_PER_SC = E // NC
E_PER_TILE = E_PER_SC // NS
CHUNK = 80
NCHUNK = E_PER_TILE // CHUNK
ROWS_PER_TILE = 624
TAIL0 = NS * ROWS_PER_TILE
TAILN = N - TAIL0
DEGW = 128


def _sliced_copy(src_ref, dst_ref, s):
    r0 = s * ROWS_PER_TILE
    pltpu.sync_copy(src_ref.at[pl.ds(r0, ROWS_PER_TILE)],
                    dst_ref.at[pl.ds(r0, ROWS_PER_TILE)])
    @pl.when(s == NS - 1)
    def _():
        pltpu.sync_copy(src_ref.at[pl.ds(TAIL0, TAILN)],
                        dst_ref.at[pl.ds(TAIL0, TAILN)])


def _drain_copy(acc_sh, out_hbm, c, s):
    r0 = s * ROWS_PER_TILE
    pltpu.sync_copy(acc_sh.at[pl.ds(r0, ROWS_PER_TILE)],
                    out_hbm.at[c, pl.ds(r0, ROWS_PER_TILE)])
    @pl.when(s == NS - 1)
    def _():
        pltpu.sync_copy(acc_sh.at[pl.ds(TAIL0, TAILN)],
                        out_hbm.at[c, pl.ds(TAIL0, TAILN)])

_mesh = plsc.VectorSubcoreMesh(
    core_axis_name="c", subcore_axis_name="s", num_cores=NC, num_subcores=NS)


@functools.partial(
    pl.kernel,
    out_type=jax.ShapeDtypeStruct((NC, N, DEGW), jnp.float32),
    mesh=_mesh,
    scratch_types=[
        pltpu.VMEM_SHARED((N, DEGW), jnp.float32),
        pltpu.VMEM((NCHUNK, CHUNK), jnp.int32),
        pltpu.VMEM((CHUNK, DEGW), jnp.float32),
    ],
)
def _deg_kernel(dst2d_hbm, zeros_hbm, ones_hbm, out_hbm, acc_sh, didx_v, ones_v):
    c = lax.axis_index("c")
    s = lax.axis_index("s")
    _sliced_copy(zeros_hbm, acc_sh, s)
    pltpu.sync_copy(dst2d_hbm.at[c * NS + s], didx_v)
    pltpu.sync_copy(ones_hbm, ones_v)
    plsc.subcore_barrier()

    def body(k, carry):
        pltpu.sync_copy(ones_v, acc_sh.at[didx_v.at[k]], add=True)
        return carry
    lax.fori_loop(0, NCHUNK, body, 0)
    plsc.subcore_barrier()
    _drain_copy(acc_sh, out_hbm, c, s)


@functools.partial(
    pl.kernel,
    out_type=jax.ShapeDtypeStruct((NC, N, H), jnp.float32),
    mesh=_mesh,
    scratch_types=[
        pltpu.VMEM_SHARED((N, H), jnp.float32),
        pltpu.VMEM((NCHUNK, CHUNK), jnp.int32),
        pltpu.VMEM((NCHUNK, CHUNK), jnp.int32),
        pltpu.VMEM((CHUNK, H), jnp.float32),
    ],
)
def _edge_kernel(g_hbm, src2d_hbm, dst2d_hbm, zeros_hbm, out_hbm,
                 acc_sh, sidx_v, didx_v, rows_v):
    c = lax.axis_index("c")
    s = lax.axis_index("s")
    @pl.when(c == 0)
    def _():
        _sliced_copy(g_hbm, acc_sh, s)
    @pl.when(c != 0)
    def _():
        _sliced_copy(zeros_hbm, acc_sh, s)
    pltpu.sync_copy(src2d_hbm.at[c * NS + s], sidx_v)
    pltpu.sync_copy(dst2d_hbm.at[c * NS + s], didx_v)
    plsc.subcore_barrier()

    def body(k, carry):
        pltpu.sync_copy(g_hbm.at[sidx_v.at[k]], rows_v)
        pltpu.sync_copy(rows_v, acc_sh.at[didx_v.at[k]], add=True)
        return carry
    lax.fori_loop(0, NCHUNK, body, 0)
    plsc.subcore_barrier()
    _drain_copy(acc_sh, out_hbm, c, s)


def _dinv_of(dp_ref):
    deg = dp_ref[0, :, 0:1] + dp_ref[1, :, 0:1] + 1.0
    return lax.rsqrt(deg)


def _mm_body(x_ref, w_ref, o_ref):
    o_ref[...] = jnp.dot(x_ref[...], w_ref[...],
                         preferred_element_type=jnp.float32)


def _scale_body(h_ref, dp_ref, o_ref):
    o_ref[...] = h_ref[...] * _dinv_of(dp_ref)


def _mid_body(sp_ref, dp_ref, w_ref, b_ref, o_ref):
    dinv = _dinv_of(dp_ref)
    x1 = (sp_ref[0] + sp_ref[1]) * dinv + b_ref[...]
    h2 = jnp.dot(x1, w_ref[...], preferred_element_type=jnp.float32)
    o_ref[...] = h2 * dinv


def _post_body(sp_ref, dp_ref, b_ref, o_ref):
    o_ref[...] = (sp_ref[0] + sp_ref[1]) * _dinv_of(dp_ref) + b_ref[...]


def _tc(body, out_shape, *args):
    return pl.pallas_call(
        body, out_shape=jax.ShapeDtypeStruct(out_shape, jnp.float32))(*args)


def kernel(edge_index, users_w, items_w, W1, b1, W2, b2):
    src2d = edge_index[0].astype(jnp.int32).reshape(NC * NS, NCHUNK, CHUNK)
    dst2d = edge_index[1].astype(jnp.int32).reshape(NC * NS, NCHUNK, CHUNK)
    x = jnp.concatenate([users_w, items_w], axis=0)
    zeros_nh = jnp.zeros((N, H), jnp.float32)
    ones_cd = jnp.ones((CHUNK, DEGW), jnp.float32)
    b1r = b1.reshape(1, H)
    b2r = b2.reshape(1, H)

    deg_parts = _deg_kernel(dst2d, zeros_nh, ones_cd)
    h1 = _tc(_mm_body, (N, H), x, W1)
    g1 = _tc(_scale_body, (N, H), h1, deg_parts)
    s1 = _edge_kernel(g1, src2d, dst2d, zeros_nh)
    g2 = _tc(_mid_body, (N, H), s1, deg_parts, W2, b1r)
    s2 = _edge_kernel(g2, src2d, dst2d, zeros_nh)
    x2 = _tc(_post_body, (N, H), s2, deg_parts, b2r)

    return (x2[:NUSERS], users_w, x2[NUSERS:], items_w)


# speedup vs baseline: 22.3708x; 1.3775x over previous
import functools

import jax
import jax.numpy as jnp
from jax import lax
from jax.experimental import pallas as pl
from jax.experimental.pallas import tpu as pltpu
from jax.experimental.pallas import tpu_sc as plsc

N = 10000
H = 128
E = 320000
NUSERS = 4000

NC = 2
NS = 16
E_PER_SC = E // NC
E_PER_TILE = E_PER_SC // NS
CHUNK = 80
NCHUNK = E_PER_TILE // CHUNK
NBLK = 5
BLK = NCHUNK // NBLK
ROWS_PER_TILE = 624
TAIL0 = NS * ROWS_PER_TILE
TAILN = N - TAIL0
DEGW = 128


def _sliced_copy(src_ref, dst_ref, s):
    r0 = s * ROWS_PER_TILE
    pltpu.sync_copy(src_ref.at[pl.ds(r0, ROWS_PER_TILE)],
                    dst_ref.at[pl.ds(r0, ROWS_PER_TILE)])
    @pl.when(s == NS - 1)
    def _():
        pltpu.sync_copy(src_ref.at[pl.ds(TAIL0, TAILN)],
                        dst_ref.at[pl.ds(TAIL0, TAILN)])


def _drain_copy(acc_sh, out_hbm, c, s):
    r0 = s * ROWS_PER_TILE
    pltpu.sync_copy(acc_sh.at[pl.ds(r0, ROWS_PER_TILE)],
                    out_hbm.at[c, pl.ds(r0, ROWS_PER_TILE)])
    @pl.when(s == NS - 1)
    def _():
        pltpu.sync_copy(acc_sh.at[pl.ds(TAIL0, TAILN)],
                        out_hbm.at[c, pl.ds(TAIL0, TAILN)])

_mesh = plsc.VectorSubcoreMesh(
    core_axis_name="c", subcore_axis_name="s", num_cores=NC, num_subcores=NS)


@functools.partial(
    pl.kernel,
    out_type=jax.ShapeDtypeStruct((NC, N, DEGW), jnp.float32),
    mesh=_mesh,
    scratch_types=[
        pltpu.VMEM_SHARED((N, DEGW), jnp.float32),
        pltpu.VMEM((BLK, CHUNK), jnp.int32),
        pltpu.VMEM((CHUNK, DEGW), jnp.float32),
    ],
)
def _deg_kernel(dst2d_hbm, zeros_hbm, ones_hbm, out_hbm, acc_sh, didx_v, ones_v):
    c = lax.axis_index("c")
    s = lax.axis_index("s")
    t = c * NS + s
    _sliced_copy(zeros_hbm, acc_sh, s)
    pltpu.sync_copy(ones_hbm, ones_v)
    plsc.subcore_barrier()

    def blk_body(b, carry):
        pltpu.sync_copy(dst2d_hbm.at[t * NBLK + b], didx_v)

        def body(k, cc):
            pltpu.sync_copy(ones_v, acc_sh.at[didx_v.at[k]], add=True)
            return cc
        lax.fori_loop(0, BLK, body, 0)
        return carry
    lax.fori_loop(0, NBLK, blk_body, 0)
    plsc.subcore_barrier()
    _drain_copy(acc_sh, out_hbm, c, s)


@functools.partial(
    pl.kernel,
    out_type=jax.ShapeDtypeStruct((NC, N, H), jnp.float32),
    mesh=_mesh,
    scratch_types=[
        pltpu.VMEM_SHARED((N, H), jnp.float32),
        pltpu.VMEM((BLK, CHUNK), jnp.int32),
        pltpu.VMEM((BLK, CHUNK), jnp.int32),
        pltpu.VMEM((CHUNK, H), jnp.float32),
        pltpu.VMEM((CHUNK, H), jnp.float32),
        pltpu.SemaphoreType.DMA,
        pltpu.SemaphoreType.DMA,
    ],
)
def _edge_kernel(g_hbm, src2d_hbm, dst2d_hbm, zeros_hbm, out_hbm,
                 acc_sh, sidx_v, didx_v, rows0_v, rows1_v, sem0, sem1):
    c = lax.axis_index("c")
    s = lax.axis_index("s")
    t = c * NS + s
    @pl.when(c == 0)
    def _():
        _sliced_copy(g_hbm, acc_sh, s)
    @pl.when(c != 0)
    def _():
        _sliced_copy(zeros_hbm, acc_sh, s)
    plsc.subcore_barrier()

    def blk_body(b, carry):
        pltpu.sync_copy(src2d_hbm.at[t * NBLK + b], sidx_v)
        pltpu.sync_copy(dst2d_hbm.at[t * NBLK + b], didx_v)
        pltpu.async_copy(g_hbm.at[sidx_v.at[0]], rows0_v, sem0)

        def body(i, cc):
            k = 2 * i
            pltpu.async_copy(g_hbm.at[sidx_v.at[k + 1]], rows1_v, sem1)
            pltpu.make_async_copy(g_hbm.at[sidx_v.at[k]], rows0_v, sem0).wait()
            pltpu.sync_copy(rows0_v, acc_sh.at[didx_v.at[k]], add=True)
            pltpu.async_copy(g_hbm.at[sidx_v.at[k + 2]], rows0_v, sem0)
            pltpu.make_async_copy(g_hbm.at[sidx_v.at[k + 1]], rows1_v,
                                  sem1).wait()
            pltpu.sync_copy(rows1_v, acc_sh.at[didx_v.at[k + 1]], add=True)
            return cc
        lax.fori_loop(0, (BLK - 1) // 2, body, 0)

        pltpu.make_async_copy(g_hbm.at[sidx_v.at[BLK - 1]], rows0_v,
                              sem0).wait()
        pltpu.sync_copy(rows0_v, acc_sh.at[didx_v.at[BLK - 1]], add=True)
        return carry
    lax.fori_loop(0, NBLK, blk_body, 0)
    plsc.subcore_barrier()
    _drain_copy(acc_sh, out_hbm, c, s)


def _dinv_of(dp_ref):
    deg = dp_ref[0, :, 0:1] + dp_ref[1, :, 0:1] + 1.0
    return lax.rsqrt(deg)


def _mm_body(x_ref, w_ref, o_ref):
    o_ref[...] = jnp.dot(x_ref[...], w_ref[...],
                         preferred_element_type=jnp.float32)


def _scale_body(h_ref, dp_ref, o_ref):
    o_ref[...] = h_ref[...] * _dinv_of(dp_ref)


def _mid_body(sp_ref, dp_ref, w_ref, b_ref, o_ref):
    dinv = _dinv_of(dp_ref)
    x1 = (sp_ref[0] + sp_ref[1]) * dinv + b_ref[...]
    h2 = jnp.dot(x1, w_ref[...], preferred_element_type=jnp.float32)
    o_ref[...] = h2 * dinv


def _post_body(sp_ref, dp_ref, b_ref, o_ref):
    o_ref[...] = (sp_ref[0] + sp_ref[1]) * _dinv_of(dp_ref) + b_ref[...]


def _tc(body, out_shape, *args):
    return pl.pallas_call(
        body, out_shape=jax.ShapeDtypeStruct(out_shape, jnp.float32))(*args)


def kernel(edge_index, users_w, items_w, W1, b1, W2, b2):
    src2d = edge_index[0].astype(jnp.int32).reshape(NC * NS * NBLK, BLK, CHUNK)
    dst2d = edge_index[1].astype(jnp.int32).reshape(NC * NS * NBLK, BLK, CHUNK)
    x = jnp.concatenate([users_w, items_w], axis=0)
    zeros_nh = jnp.zeros((N, H), jnp.float32)
    ones_cd = jnp.ones((CHUNK, DEGW), jnp.float32)
    b1r = b1.reshape(1, H)
    b2r = b2.reshape(1, H)

    deg_parts = _deg_kernel(dst2d, zeros_nh, ones_cd)
    h1 = _tc(_mm_body, (N, H), x, W1)
    g1 = _tc(_scale_body, (N, H), h1, deg_parts)
    s1 = _edge_kernel(g1, src2d, dst2d, zeros_nh)
    g2 = _tc(_mid_body, (N, H), s1, deg_parts, W2, b1r)
    s2 = _edge_kernel(g2, src2d, dst2d, zeros_nh)
    x2 = _tc(_post_body, (N, H), s2, deg_parts, b2r)

    return (x2[:NUSERS], users_w, x2[NUSERS:], items_w)


# speedup vs baseline: 24.7548x; 1.1066x over previous
import functools

import jax
import jax.numpy as jnp
from jax import lax
from jax.experimental import pallas as pl
from jax.experimental.pallas import tpu as pltpu
from jax.experimental.pallas import tpu_sc as plsc

N = 10000
H = 128
E = 320000
NUSERS = 4000

NC = 2
NS = 16
E_PER_SC = E // NC
E_PER_TILE = E_PER_SC // NS
CHUNK = 80
NCHUNK = E_PER_TILE // CHUNK
NBLK = 5
BLK = NCHUNK // NBLK
NBUF = 3
ROWS_PER_TILE = 624
TAIL0 = NS * ROWS_PER_TILE
TAILN = N - TAIL0
DEGW = 128


def _sliced_copy(src_ref, dst_ref, s):
    r0 = s * ROWS_PER_TILE
    pltpu.sync_copy(src_ref.at[pl.ds(r0, ROWS_PER_TILE)],
                    dst_ref.at[pl.ds(r0, ROWS_PER_TILE)])
    @pl.when(s == NS - 1)
    def _():
        pltpu.sync_copy(src_ref.at[pl.ds(TAIL0, TAILN)],
                        dst_ref.at[pl.ds(TAIL0, TAILN)])


def _drain_copy(acc_sh, out_hbm, c, s):
    r0 = s * ROWS_PER_TILE
    pltpu.sync_copy(acc_sh.at[pl.ds(r0, ROWS_PER_TILE)],
                    out_hbm.at[c, pl.ds(r0, ROWS_PER_TILE)])
    @pl.when(s == NS - 1)
    def _():
        pltpu.sync_copy(acc_sh.at[pl.ds(TAIL0, TAILN)],
                        out_hbm.at[c, pl.ds(TAIL0, TAILN)])

_mesh = plsc.VectorSubcoreMesh(
    core_axis_name="c", subcore_axis_name="s", num_cores=NC, num_subcores=NS)


@functools.partial(
    pl.kernel,
    out_type=jax.ShapeDtypeStruct((NC, N, DEGW), jnp.float32),
    mesh=_mesh,
    scratch_types=[
        pltpu.VMEM_SHARED((N, DEGW), jnp.float32),
        pltpu.VMEM((BLK, CHUNK), jnp.int32),
        pltpu.VMEM((CHUNK, DEGW), jnp.float32),
    ],
)
def _deg_kernel(dst2d_hbm, zeros_hbm, ones_hbm, out_hbm, acc_sh, didx_v, ones_v):
    c = lax.axis_index("c")
    s = lax.axis_index("s")
    t = c * NS + s
    _sliced_copy(zeros_hbm, acc_sh, s)
    pltpu.sync_copy(ones_hbm, ones_v)
    plsc.subcore_barrier()

    def blk_body(b, carry):
        pltpu.sync_copy(dst2d_hbm.at[t * NBLK + b], didx_v)

        def body(k, cc):
            pltpu.sync_copy(ones_v, acc_sh.at[didx_v.at[k]], add=True)
            return cc
        lax.fori_loop(0, BLK, body, 0)
        return carry
    lax.fori_loop(0, NBLK, blk_body, 0)
    plsc.subcore_barrier()
    _drain_copy(acc_sh, out_hbm, c, s)


@functools.partial(
    pl.kernel,
    out_type=jax.ShapeDtypeStruct((NC, N, H), jnp.float32),
    mesh=_mesh,
    scratch_types=[
        pltpu.VMEM_SHARED((N, H), jnp.float32),
        pltpu.VMEM((BLK, CHUNK), jnp.int32),
        pltpu.VMEM((BLK, CHUNK), jnp.int32),
        pltpu.VMEM((NBUF, CHUNK, H), jnp.float32),
        pltpu.SemaphoreType.DMA((NBUF,)),
    ],
)
def _edge_kernel(g_hbm, src2d_hbm, dst2d_hbm, zeros_hbm, out_hbm,
                 acc_sh, sidx_v, didx_v, rows_v, gsem):
    c = lax.axis_index("c")
    s = lax.axis_index("s")
    t = c * NS + s
    @pl.when(c == 0)
    def _():
        _sliced_copy(g_hbm, acc_sh, s)
    @pl.when(c != 0)
    def _():
        _sliced_copy(zeros_hbm, acc_sh, s)
    plsc.subcore_barrier()

    def blk_body(b, carry):
        pltpu.sync_copy(src2d_hbm.at[t * NBLK + b], sidx_v)
        pltpu.sync_copy(dst2d_hbm.at[t * NBLK + b], didx_v)
        for j in range(NBUF):
            pltpu.async_copy(g_hbm.at[sidx_v.at[j]], rows_v.at[j], gsem.at[j])

        def body(k, cc):
            j = lax.rem(k, NBUF)
            pltpu.make_async_copy(g_hbm.at[sidx_v.at[k]], rows_v.at[j],
                                  gsem.at[j]).wait()
            pltpu.sync_copy(rows_v.at[j], acc_sh.at[didx_v.at[k]], add=True)
            pltpu.async_copy(g_hbm.at[sidx_v.at[k + NBUF]], rows_v.at[j],
                             gsem.at[j])
            return cc
        lax.fori_loop(0, BLK - NBUF, body, 0)

        def tail(k, cc):
            j = lax.rem(k, NBUF)
            pltpu.make_async_copy(g_hbm.at[sidx_v.at[k]], rows_v.at[j],
                                  gsem.at[j]).wait()
            pltpu.sync_copy(rows_v.at[j], acc_sh.at[didx_v.at[k]], add=True)
            return cc
        lax.fori_loop(BLK - NBUF, BLK, tail, 0)
        return carry
    lax.fori_loop(0, NBLK, blk_body, 0)
    plsc.subcore_barrier()
    _drain_copy(acc_sh, out_hbm, c, s)


def _dinv_of(dp_ref):
    deg = dp_ref[0, :, 0:1] + dp_ref[1, :, 0:1] + 1.0
    return lax.rsqrt(deg)


def _mm_body(x_ref, w_ref, o_ref):
    o_ref[...] = jnp.dot(x_ref[...], w_ref[...],
                         preferred_element_type=jnp.float32)


def _scale_body(h_ref, dp_ref, o_ref):
    o_ref[...] = h_ref[...] * _dinv_of(dp_ref)


def _mid_body(sp_ref, dp_ref, w_ref, b_ref, o_ref):
    dinv = _dinv_of(dp_ref)
    x1 = (sp_ref[0] + sp_ref[1]) * dinv + b_ref[...]
    h2 = jnp.dot(x1, w_ref[...], preferred_element_type=jnp.float32)
    o_ref[...] = h2 * dinv


def _post_body(sp_ref, dp_ref, b_ref, o_ref):
    o_ref[...] = (sp_ref[0] + sp_ref[1]) * _dinv_of(dp_ref) + b_ref[...]


def _tc(body, out_shape, *args):
    return pl.pallas_call(
        body, out_shape=jax.ShapeDtypeStruct(out_shape, jnp.float32))(*args)


def kernel(edge_index, users_w, items_w, W1, b1, W2, b2):
    src2d = edge_index[0].astype(jnp.int32).reshape(NC * NS * NBLK, BLK, CHUNK)
    dst2d = edge_index[1].astype(jnp.int32).reshape(NC * NS * NBLK, BLK, CHUNK)
    x = jnp.concatenate([users_w, items_w], axis=0)
    zeros_nh = jnp.zeros((N, H), jnp.float32)
    ones_cd = jnp.ones((CHUNK, DEGW), jnp.float32)
    b1r = b1.reshape(1, H)
    b2r = b2.reshape(1, H)

    deg_parts = _deg_kernel(dst2d, zeros_nh, ones_cd)
    h1 = _tc(_mm_body, (N, H), x, W1)
    g1 = _tc(_scale_body, (N, H), h1, deg_parts)
    s1 = _edge_kernel(g1, src2d, dst2d, zeros_nh)
    g2 = _tc(_mid_body, (N, H), s1, deg_parts, W2, b1r)
    s2 = _edge_kernel(g2, src2d, dst2d, zeros_nh)
    x2 = _tc(_post_body, (N, H), s2, deg_parts, b2r)

    return (x2[:NUSERS], users_w, x2[NUSERS:], items_w)


# speedup vs baseline: 24.9323x; 1.0072x over previous
import functools

import jax
import jax.numpy as jnp
from jax import lax
from jax.experimental import pallas as pl
from jax.experimental.pallas import tpu as pltpu
from jax.experimental.pallas import tpu_sc as plsc

N = 10000
H = 128
E = 320000
NUSERS = 4000

NC = 2
NS = 16
E_PER_SC = E // NC
E_PER_TILE = E_PER_SC // NS
CHUNK = 80
NCHUNK = E_PER_TILE // CHUNK
NBLK = 5
BLK = NCHUNK // NBLK
NBUF = 4
ROWS_PER_TILE = 624
TAIL0 = NS * ROWS_PER_TILE
TAILN = N - TAIL0
DEGW = 128


def _sliced_copy(src_ref, dst_ref, s):
    r0 = s * ROWS_PER_TILE
    pltpu.sync_copy(src_ref.at[pl.ds(r0, ROWS_PER_TILE)],
                    dst_ref.at[pl.ds(r0, ROWS_PER_TILE)])
    @pl.when(s == NS - 1)
    def _():
        pltpu.sync_copy(src_ref.at[pl.ds(TAIL0, TAILN)],
                        dst_ref.at[pl.ds(TAIL0, TAILN)])


def _drain_copy(acc_sh, out_hbm, c, s):
    r0 = s * ROWS_PER_TILE
    pltpu.sync_copy(acc_sh.at[pl.ds(r0, ROWS_PER_TILE)],
                    out_hbm.at[c, pl.ds(r0, ROWS_PER_TILE)])
    @pl.when(s == NS - 1)
    def _():
        pltpu.sync_copy(acc_sh.at[pl.ds(TAIL0, TAILN)],
                        out_hbm.at[c, pl.ds(TAIL0, TAILN)])

_mesh = plsc.VectorSubcoreMesh(
    core_axis_name="c", subcore_axis_name="s", num_cores=NC, num_subcores=NS)


@functools.partial(
    pl.kernel,
    out_type=jax.ShapeDtypeStruct((NC, N, DEGW), jnp.float32),
    mesh=_mesh,
    scratch_types=[
        pltpu.VMEM_SHARED((N, DEGW), jnp.float32),
        pltpu.VMEM((BLK, CHUNK), jnp.int32),
        pltpu.VMEM((CHUNK, DEGW), jnp.float32),
    ],
)
def _deg_kernel(dst2d_hbm, zeros_hbm, ones_hbm, out_hbm, acc_sh, didx_v, ones_v):
    c = lax.axis_index("c")
    s = lax.axis_index("s")
    t = c * NS + s
    _sliced_copy(zeros_hbm, acc_sh, s)
    pltpu.sync_copy(ones_hbm, ones_v)
    plsc.subcore_barrier()

    def blk_body(b, carry):
        pltpu.sync_copy(dst2d_hbm.at[t * NBLK + b], didx_v)

        def body(k, cc):
            pltpu.sync_copy(ones_v, acc_sh.at[didx_v.at[k]], add=True)
            return cc
        lax.fori_loop(0, BLK, body, 0)
        return carry
    lax.fori_loop(0, NBLK, blk_body, 0)
    plsc.subcore_barrier()
    _drain_copy(acc_sh, out_hbm, c, s)


@functools.partial(
    pl.kernel,
    out_type=jax.ShapeDtypeStruct((NC, N, H), jnp.float32),
    mesh=_mesh,
    scratch_types=[
        pltpu.VMEM_SHARED((N, H), jnp.float32),
        pltpu.VMEM((BLK, CHUNK), jnp.int32),
        pltpu.VMEM((BLK, CHUNK), jnp.int32),
        pltpu.VMEM((NBUF, CHUNK, H), jnp.float32),
        pltpu.SemaphoreType.DMA((NBUF,)),
    ],
)
def _edge_kernel(g_hbm, src2d_hbm, dst2d_hbm, zeros_hbm, out_hbm,
                 acc_sh, sidx_v, didx_v, rows_v, gsem):
    c = lax.axis_index("c")
    s = lax.axis_index("s")
    t = c * NS + s
    @pl.when(c == 0)
    def _():
        _sliced_copy(g_hbm, acc_sh, s)
    @pl.when(c != 0)
    def _():
        _sliced_copy(zeros_hbm, acc_sh, s)
    plsc.subcore_barrier()

    def blk_body(b, carry):
        pltpu.sync_copy(src2d_hbm.at[t * NBLK + b], sidx_v)
        pltpu.sync_copy(dst2d_hbm.at[t * NBLK + b], didx_v)
        for j in range(NBUF):
            pltpu.async_copy(g_hbm.at[sidx_v.at[j]], rows_v.at[j], gsem.at[j])

        def body(k, cc):
            j = lax.rem(k, NBUF)
            pltpu.make_async_copy(g_hbm.at[sidx_v.at[k]], rows_v.at[j],
                                  gsem.at[j]).wait()
            pltpu.sync_copy(rows_v.at[j], acc_sh.at[didx_v.at[k]], add=True)
            pltpu.async_copy(g_hbm.at[sidx_v.at[k + NBUF]], rows_v.at[j],
                             gsem.at[j])
            return cc
        lax.fori_loop(0, BLK - NBUF, body, 0)

        def tail(k, cc):
            j = lax.rem(k, NBUF)
            pltpu.make_async_copy(g_hbm.at[sidx_v.at[k]], rows_v.at[j],
                                  gsem.at[j]).wait()
            pltpu.sync_copy(rows_v.at[j], acc_sh.at[didx_v.at[k]], add=True)
            return cc
        lax.fori_loop(BLK - NBUF, BLK, tail, 0)
        return carry
    lax.fori_loop(0, NBLK, blk_body, 0)
    plsc.subcore_barrier()
    _drain_copy(acc_sh, out_hbm, c, s)


def _dinv_of(dp_ref):
    deg = dp_ref[0, :, 0:1] + dp_ref[1, :, 0:1] + 1.0
    return lax.rsqrt(deg)


def _mm_body(x_ref, w_ref, o_ref):
    o_ref[...] = jnp.dot(x_ref[...], w_ref[...],
                         preferred_element_type=jnp.float32)


def _scale_body(h_ref, dp_ref, o_ref):
    o_ref[...] = h_ref[...] * _dinv_of(dp_ref)


def _mid_body(sp_ref, dp_ref, w_ref, b_ref, o_ref):
    dinv = _dinv_of(dp_ref)
    x1 = (sp_ref[0] + sp_ref[1]) * dinv + b_ref[...]
    h2 = jnp.dot(x1, w_ref[...], preferred_element_type=jnp.float32)
    o_ref[...] = h2 * dinv


def _post_body(sp_ref, dp_ref, b_ref, o_ref):
    o_ref[...] = (sp_ref[0] + sp_ref[1]) * _dinv_of(dp_ref) + b_ref[...]


def _tc(body, out_shape, *args):
    return pl.pallas_call(
        body, out_shape=jax.ShapeDtypeStruct(out_shape, jnp.float32))(*args)


def kernel(edge_index, users_w, items_w, W1, b1, W2, b2):
    src2d = edge_index[0].astype(jnp.int32).reshape(NC * NS * NBLK, BLK, CHUNK)
    dst2d = edge_index[1].astype(jnp.int32).reshape(NC * NS * NBLK, BLK, CHUNK)
    x = jnp.concatenate([users_w, items_w], axis=0)
    zeros_nh = jnp.zeros((N, H), jnp.float32)
    ones_cd = jnp.ones((CHUNK, DEGW), jnp.float32)
    b1r = b1.reshape(1, H)
    b2r = b2.reshape(1, H)

    deg_parts = _deg_kernel(dst2d, zeros_nh, ones_cd)
    h1 = _tc(_mm_body, (N, H), x, W1)
    g1 = _tc(_scale_body, (N, H), h1, deg_parts)
    s1 = _edge_kernel(g1, src2d, dst2d, zeros_nh)
    g2 = _tc(_mid_body, (N, H), s1, deg_parts, W2, b1r)
    s2 = _edge_kernel(g2, src2d, dst2d, zeros_nh)
    x2 = _tc(_post_body, (N, H), s2, deg_parts, b2r)

    return (x2[:NUSERS], users_w, x2[NUSERS:], items_w)


# speedup vs baseline: 25.0032x; 1.0028x over previous
import functools

import jax
import jax.numpy as jnp
from jax import lax
from jax.experimental import pallas as pl
from jax.experimental.pallas import tpu as pltpu
from jax.experimental.pallas import tpu_sc as plsc

N = 10000
H = 128
E = 320000
NUSERS = 4000

NC = 2
NS = 16
E_PER_SC = E // NC
E_PER_TILE = E_PER_SC // NS
CHUNK = 80
NCHUNK = E_PER_TILE // CHUNK
NBLK = 5
BLK = NCHUNK // NBLK
NBUF = 4
ROWS_PER_TILE = 624
TAIL0 = NS * ROWS_PER_TILE
TAILN = N - TAIL0
DEGW = 128


def _sliced_copy(src_ref, dst_ref, s):
    r0 = s * ROWS_PER_TILE
    pltpu.sync_copy(src_ref.at[pl.ds(r0, ROWS_PER_TILE)],
                    dst_ref.at[pl.ds(r0, ROWS_PER_TILE)])
    @pl.when(s == NS - 1)
    def _():
        pltpu.sync_copy(src_ref.at[pl.ds(TAIL0, TAILN)],
                        dst_ref.at[pl.ds(TAIL0, TAILN)])


def _drain_copy(acc_sh, out_hbm, c, s):
    r0 = s * ROWS_PER_TILE
    pltpu.sync_copy(acc_sh.at[pl.ds(r0, ROWS_PER_TILE)],
                    out_hbm.at[c, pl.ds(r0, ROWS_PER_TILE)])
    @pl.when(s == NS - 1)
    def _():
        pltpu.sync_copy(acc_sh.at[pl.ds(TAIL0, TAILN)],
                        out_hbm.at[c, pl.ds(TAIL0, TAILN)])

_mesh = plsc.VectorSubcoreMesh(
    core_axis_name="c", subcore_axis_name="s", num_cores=NC, num_subcores=NS)


@functools.partial(
    pl.kernel,
    out_type=jax.ShapeDtypeStruct((NC, N, DEGW), jnp.float32),
    mesh=_mesh,
    scratch_types=[
        pltpu.VMEM_SHARED((N, DEGW), jnp.float32),
        pltpu.VMEM((BLK, CHUNK), jnp.int32),
        pltpu.VMEM((CHUNK, DEGW), jnp.float32),
    ],
)
def _deg_kernel(dst2d_hbm, zeros_hbm, ones_hbm, out_hbm, acc_sh, didx_v, ones_v):
    c = lax.axis_index("c")
    s = lax.axis_index("s")
    t = c * NS + s
    _sliced_copy(zeros_hbm, acc_sh, s)
    pltpu.sync_copy(ones_hbm, ones_v)
    plsc.subcore_barrier()

    def blk_body(b, carry):
        pltpu.sync_copy(dst2d_hbm.at[t * NBLK + b], didx_v)

        def body(k, cc):
            pltpu.sync_copy(ones_v, acc_sh.at[didx_v.at[k]], add=True)
            return cc
        lax.fori_loop(0, BLK, body, 0)
        return carry
    lax.fori_loop(0, NBLK, blk_body, 0)
    plsc.subcore_barrier()
    _drain_copy(acc_sh, out_hbm, c, s)


@functools.partial(
    pl.kernel,
    out_type=jax.ShapeDtypeStruct((NC, N, H), jnp.float32),
    mesh=_mesh,
    scratch_types=[
        pltpu.VMEM_SHARED((N, H), jnp.float32),
        pltpu.VMEM((BLK, CHUNK), jnp.int32),
        pltpu.VMEM((BLK, CHUNK), jnp.int32),
        pltpu.VMEM((NBUF, CHUNK, H), jnp.float32),
        pltpu.SemaphoreType.DMA((NBUF,)),
        pltpu.SemaphoreType.DMA((NBUF,)),
    ],
)
def _edge_kernel(g_hbm, src2d_hbm, dst2d_hbm, zeros_hbm, out_hbm,
                 acc_sh, sidx_v, didx_v, rows_v, gsem, ssem):
    c = lax.axis_index("c")
    s = lax.axis_index("s")
    t = c * NS + s
    @pl.when(c == 0)
    def _():
        _sliced_copy(g_hbm, acc_sh, s)
    @pl.when(c != 0)
    def _():
        _sliced_copy(zeros_hbm, acc_sh, s)
    plsc.subcore_barrier()

    def blk_body(b, carry):
        pltpu.sync_copy(src2d_hbm.at[t * NBLK + b], sidx_v)
        pltpu.sync_copy(dst2d_hbm.at[t * NBLK + b], didx_v)
        for j in range(NBUF):
            pltpu.async_copy(g_hbm.at[sidx_v.at[j]], rows_v.at[j], gsem.at[j])

        pltpu.make_async_copy(g_hbm.at[sidx_v.at[0]], rows_v.at[0],
                              gsem.at[0]).wait()
        pltpu.async_copy(rows_v.at[0], acc_sh.at[didx_v.at[0]], ssem.at[0],
                         add=True)

        def body(k, cc):
            jp = lax.rem(k - 1, NBUF)
            pltpu.make_async_copy(rows_v.at[jp], acc_sh.at[didx_v.at[k - 1]],
                                  ssem.at[jp]).wait()
            pltpu.async_copy(g_hbm.at[sidx_v.at[k - 1 + NBUF]], rows_v.at[jp],
                             gsem.at[jp])
            j = lax.rem(k, NBUF)
            pltpu.make_async_copy(g_hbm.at[sidx_v.at[k]], rows_v.at[j],
                                  gsem.at[j]).wait()
            pltpu.async_copy(rows_v.at[j], acc_sh.at[didx_v.at[k]],
                             ssem.at[j], add=True)
            return cc
        lax.fori_loop(1, BLK - NBUF + 1, body, 0)

        def tail(k, cc):
            j = lax.rem(k, NBUF)
            pltpu.make_async_copy(g_hbm.at[sidx_v.at[k]], rows_v.at[j],
                                  gsem.at[j]).wait()
            pltpu.async_copy(rows_v.at[j], acc_sh.at[didx_v.at[k]],
                             ssem.at[j], add=True)
            return cc
        lax.fori_loop(BLK - NBUF + 1, BLK, tail, 0)

        def drain(i, cc):
            k = BLK - NBUF + i
            j = lax.rem(k, NBUF)
            pltpu.make_async_copy(rows_v.at[j], acc_sh.at[didx_v.at[k]],
                                  ssem.at[j]).wait()
            return cc
        lax.fori_loop(0, NBUF, drain, 0)
        return carry
    lax.fori_loop(0, NBLK, blk_body, 0)
    plsc.subcore_barrier()
    _drain_copy(acc_sh, out_hbm, c, s)


def _dinv_of(dp_ref):
    deg = dp_ref[0, :, 0:1] + dp_ref[1, :, 0:1] + 1.0
    return lax.rsqrt(deg)


def _mm_body(x_ref, w_ref, o_ref):
    o_ref[...] = jnp.dot(x_ref[...], w_ref[...],
                         preferred_element_type=jnp.float32)


def _scale_body(h_ref, dp_ref, o_ref):
    o_ref[...] = h_ref[...] * _dinv_of(dp_ref)


def _mid_body(sp_ref, dp_ref, w_ref, b_ref, o_ref):
    dinv = _dinv_of(dp_ref)
    x1 = (sp_ref[0] + sp_ref[1]) * dinv + b_ref[...]
    h2 = jnp.dot(x1, w_ref[...], preferred_element_type=jnp.float32)
    o_ref[...] = h2 * dinv


def _post_body(sp_ref, dp_ref, b_ref, o_ref):
    o_ref[...] = (sp_ref[0] + sp_ref[1]) * _dinv_of(dp_ref) + b_ref[...]


def _tc(body, out_shape, *args):
    return pl.pallas_call(
        body, out_shape=jax.ShapeDtypeStruct(out_shape, jnp.float32))(*args)


def kernel(edge_index, users_w, items_w, W1, b1, W2, b2):
    src2d = edge_index[0].astype(jnp.int32).reshape(NC * NS * NBLK, BLK, CHUNK)
    dst2d = edge_index[1].astype(jnp.int32).reshape(NC * NS * NBLK, BLK, CHUNK)
    x = jnp.concatenate([users_w, items_w], axis=0)
    zeros_nh = jnp.zeros((N, H), jnp.float32)
    ones_cd = jnp.ones((CHUNK, DEGW), jnp.float32)
    b1r = b1.reshape(1, H)
    b2r = b2.reshape(1, H)

    deg_parts = _deg_kernel(dst2d, zeros_nh, ones_cd)
    h1 = _tc(_mm_body, (N, H), x, W1)
    g1 = _tc(_scale_body, (N, H), h1, deg_parts)
    s1 = _edge_kernel(g1, src2d, dst2d, zeros_nh)
    g2 = _tc(_mid_body, (N, H), s1, deg_parts, W2, b1r)
    s2 = _edge_kernel(g2, src2d, dst2d, zeros_nh)
    x2 = _tc(_post_body, (N, H), s2, deg_parts, b2r)

    return (x2[:NUSERS], users_w, x2[NUSERS:], items_w)
